# baseline (speedup 1.0000x reference)
import functools

import jax
import jax.numpy as jnp
from jax import lax
from jax.experimental import pallas as pl
from jax.experimental.pallas import tpu as pltpu
from jax.experimental.pallas import tpu_sc as plsc

SEQ = 200
BATCH = 1024
EMB = 64
C = 128
NLANES = 16
NW = 32
TILES_PER_POS = BATCH // C
TOTAL_TILES = SEQ * TILES_PER_POS
PER_W = TOTAL_TILES // NW
NGROUPS = C // NLANES
E_UNROLL = 4


def _emb_kernel(x_hbm, enc2_hbm, post_hbm, out_hbm,
                xrow, idx2_all, rows, posbuf, outbuf,
                gsem0, gsem1, osem0, osem1):
    cid = lax.axis_index("c")
    sid = lax.axis_index("s")
    wid = sid * 2 + cid
    tau0 = wid * PER_W
    s0 = tau0 // TILES_PER_POS

    pltpu.sync_copy(post_hbm.at[:, pl.ds(0, 128)], posbuf.at[0])
    pltpu.sync_copy(post_hbm.at[:, pl.ds(128, 128)], posbuf.at[1])

    gsems = (gsem0, gsem1)
    osems = (osem0, osem1)
    iota = lax.iota(jnp.int32, NLANES)

    def tile_coords(t):
        tau = tau0 + t
        s = tau // TILES_PER_POS
        b0 = (tau % TILES_PER_POS) * C
        return s, b0

    def fetch_idx(t, b):
        s, b0 = tile_coords(t)
        pltpu.sync_copy(x_hbm.at[s, pl.ds(b0, C)], xrow.at[b])
        for g in range(NGROUPS):
            sl = pl.ds(g * NLANES, NLANES)
            idx2_all[b, sl] = xrow[b, sl] >> 1

    def gather(t, b):
        pltpu.async_copy(enc2_hbm.at[idx2_all.at[b]], rows.at[b], gsems[b])

    def wait_gather(t, b):
        pltpu.make_async_copy(
            enc2_hbm.at[idx2_all.at[b]], rows.at[b], gsems[b]).wait()

    def store_out(t, b):
        s, b0 = tile_coords(t)
        pltpu.async_copy(outbuf.at[b], out_hbm.at[s, :, pl.ds(b0, C)],
                         osems[b])

    def wait_store(t, b):
        s, b0 = tile_coords(t)
        pltpu.make_async_copy(outbuf.at[b], out_hbm.at[s, :, pl.ds(b0, C)],
                              osems[b]).wait()

    fetch_idx(0, 0)
    gather(0, 0)
    fetch_idx(1, 1)
    gather(1, 1)

    def pair(i, carry):
        for b in range(2):
            t = 2 * i + b
            s, b0 = tile_coords(t)

            wait_gather(t, b)

            @pl.when(t >= 2)
            def _():
                wait_store(t - 2, b)

            rowv = []
            colbase = []
            for g in range(NGROUPS):
                sl = pl.ds(g * NLANES, NLANES)
                rowv.append(iota + (g * NLANES))
                colbase.append((xrow[b, sl] & 1) << 6)

            blk = s >> 7
            slocv = jnp.broadcast_to(s & 127, (NLANES,))

            def eloop(k, c2):
                for u in range(E_UNROLL):
                    e = k * E_UNROLL + u
                    p = plsc.load_gather(
                        posbuf.at[blk], [jnp.broadcast_to(e, (NLANES,)), slocv])
                    for g in range(NGROUPS):
                        v = plsc.load_gather(
                            rows.at[b], [rowv[g], colbase[g] + e])
                        outbuf[b, e, pl.ds(g * NLANES, NLANES)] = v + p
                return c2

            lax.fori_loop(0, EMB // E_UNROLL, eloop, 0)

            store_out(t, b)

            @pl.when(t + 2 < PER_W)
            def _():
                fetch_idx(t + 2, b)
                gather(t + 2, b)
        return carry

    lax.fori_loop(0, PER_W // 2, pair, 0)

    wait_store(PER_W - 2, 0)
    wait_store(PER_W - 1, 1)


def kernel(x, encoder, pos_emb):
    mesh = plsc.VectorSubcoreMesh(core_axis_name="c", subcore_axis_name="s")
    run = functools.partial(
        pl.kernel,
        mesh=mesh,
        out_type=jax.ShapeDtypeStruct((SEQ, EMB, BATCH), jnp.float32),
        scratch_types=[
            pltpu.VMEM((2, C), jnp.int32),
            pltpu.VMEM((2, C), jnp.int32),
            pltpu.VMEM((2, C, 2 * EMB), jnp.float32),
            pltpu.VMEM((2, EMB, 128), jnp.float32),
            pltpu.VMEM((2, EMB, C), jnp.float32),
            pltpu.SemaphoreType.DMA,
            pltpu.SemaphoreType.DMA,
            pltpu.SemaphoreType.DMA,
            pltpu.SemaphoreType.DMA,
        ],
        compiler_params=pltpu.CompilerParams(
            use_tc_tiling_on_sc=True, needs_layout_passes=False),
    )(_emb_kernel)
    out_t = run(x, encoder.reshape(-1, 2 * EMB), pos_emb.T)
    return jnp.swapaxes(out_t, 1, 2)


# speedup vs baseline: 1.4008x; 1.4008x over previous
import functools

import jax
import jax.numpy as jnp
from jax import lax
from jax.experimental import pallas as pl
from jax.experimental.pallas import tpu as pltpu
from jax.experimental.pallas import tpu_sc as plsc

SEQ = 200
BATCH = 1024
EMB = 64
C = 128
NLANES = 16
NW = 32
TILES_PER_POS = BATCH // C
TOTAL_TILES = SEQ * TILES_PER_POS
PER_W = TOTAL_TILES // NW
NJ = EMB // NLANES
ROW_UNROLL = 4


def _emb_kernel(x_hbm, enc_hbm, post_hbm, out_hbm,
                idx, rows, posbuf, outbuf,
                gsem0, gsem1, osem0, osem1):
    cid = lax.axis_index("c")
    sid = lax.axis_index("s")
    wid = sid * 2 + cid
    tau0 = wid * PER_W

    pltpu.sync_copy(post_hbm.at[:, pl.ds(0, 128)], posbuf.at[0])
    pltpu.sync_copy(post_hbm.at[:, pl.ds(128, 128)], posbuf.at[1])

    gsems = (gsem0, gsem1)
    osems = (osem0, osem1)
    iota = lax.iota(jnp.int32, NLANES)

    def tile_coords(t):
        tau = tau0 + t
        s = tau // TILES_PER_POS
        b0 = (tau % TILES_PER_POS) * C
        return s, b0

    def fetch_idx_and_gather(t, b):
        s, b0 = tile_coords(t)
        pltpu.sync_copy(x_hbm.at[s, pl.ds(b0, C)], idx.at[b])
        pltpu.async_copy(enc_hbm.at[idx.at[b]], rows.at[b], gsems[b])

    def wait_gather(t, b):
        pltpu.make_async_copy(
            enc_hbm.at[idx.at[b]], rows.at[b], gsems[b]).wait()

    def store_out(t, b):
        s, b0 = tile_coords(t)
        pltpu.async_copy(outbuf.at[b], out_hbm.at[s, pl.ds(b0, C)], osems[b])

    def wait_store(t, b):
        s, b0 = tile_coords(t)
        pltpu.make_async_copy(outbuf.at[b], out_hbm.at[s, pl.ds(b0, C)],
                              osems[b]).wait()

    fetch_idx_and_gather(0, 0)
    fetch_idx_and_gather(1, 1)

    def pair(i, carry):
        for b in range(2):
            t = 2 * i + b
            s, b0 = tile_coords(t)

            wait_gather(t, b)

            @pl.when(t >= 2)
            def _():
                wait_store(t - 2, b)

            blk = s >> 7
            slocv = jnp.broadcast_to(s & 127, (NLANES,))
            p = [plsc.load_gather(posbuf.at[blk], [iota + (j * NLANES), slocv])
                 for j in range(NJ)]

            def addrows(k, c2):
                for u in range(ROW_UNROLL):
                    r = k * ROW_UNROLL + u
                    for j in range(NJ):
                        sl = pl.ds(j * NLANES, NLANES)
                        outbuf[b, r, sl] = rows[b, r, sl] + p[j]
                return c2

            lax.fori_loop(0, C // ROW_UNROLL, addrows, 0)

            store_out(t, b)

            @pl.when(t + 2 < PER_W)
            def _():
                fetch_idx_and_gather(t + 2, b)
        return carry

    lax.fori_loop(0, PER_W // 2, pair, 0)

    wait_store(PER_W - 2, 0)
    wait_store(PER_W - 1, 1)


def kernel(x, encoder, pos_emb):
    mesh = plsc.VectorSubcoreMesh(core_axis_name="c", subcore_axis_name="s")
    run = functools.partial(
        pl.kernel,
        mesh=mesh,
        out_type=jax.ShapeDtypeStruct((SEQ, BATCH, EMB), jnp.float32),
        scratch_types=[
            pltpu.VMEM((2, C), jnp.int32),
            pltpu.VMEM((2, C, 2 * EMB), jnp.float32),
            pltpu.VMEM((2, EMB, 128), jnp.float32),
            pltpu.VMEM((2, C, EMB), jnp.float32),
            pltpu.SemaphoreType.DMA,
            pltpu.SemaphoreType.DMA,
            pltpu.SemaphoreType.DMA,
            pltpu.SemaphoreType.DMA,
        ],
        compiler_params=pltpu.CompilerParams(
            use_tc_tiling_on_sc=True, needs_layout_passes=False),
    )(_emb_kernel)
    enc_padded = jnp.pad(encoder, ((0, 0), (0, 2 * EMB - EMB)))
    return run(x, enc_padded, pos_emb.T)


# speedup vs baseline: 1.4052x; 1.0032x over previous
import functools

import jax
import jax.numpy as jnp
from jax import lax
from jax.experimental import pallas as pl
from jax.experimental.pallas import tpu as pltpu
from jax.experimental.pallas import tpu_sc as plsc

SEQ = 200
BATCH = 1024
EMB = 64
C = 128
NLANES = 16
NW = 32
TILES_PER_POS = BATCH // C
TOTAL_TILES = SEQ * TILES_PER_POS
PER_W = TOTAL_TILES // NW
NJ = EMB // NLANES
ROW_UNROLL = 4


def _emb_kernel(x_hbm, enc_hbm, post_hbm, out_hbm,
                idx, rows, posbuf, outbuf,
                gsem0, gsem1, osem0, osem1, isem0, isem1):
    cid = lax.axis_index("c")
    sid = lax.axis_index("s")
    wid = sid * 2 + cid
    tau0 = wid * PER_W

    pltpu.sync_copy(post_hbm.at[:, pl.ds(0, 128)], posbuf.at[0])
    pltpu.sync_copy(post_hbm.at[:, pl.ds(128, 128)], posbuf.at[1])

    gsems = (gsem0, gsem1)
    osems = (osem0, osem1)
    isems = (isem0, isem1)
    iota = lax.iota(jnp.int32, NLANES)
    erow = [iota + (j * NLANES) for j in range(NJ)]

    def tile_coords(t):
        tau = tau0 + t
        s = tau // TILES_PER_POS
        b0 = (tau % TILES_PER_POS) * C
        return s, b0

    def fetch_idx(t, b):
        s, b0 = tile_coords(t)
        pltpu.async_copy(x_hbm.at[s, pl.ds(b0, C)], idx.at[b], isems[b])

    def wait_idx(t, b):
        s, b0 = tile_coords(t)
        pltpu.make_async_copy(x_hbm.at[s, pl.ds(b0, C)], idx.at[b],
                              isems[b]).wait()

    def gather(t, b):
        pltpu.async_copy(enc_hbm.at[idx.at[b]], rows.at[b], gsems[b])

    def wait_gather(t, b):
        pltpu.make_async_copy(
            enc_hbm.at[idx.at[b]], rows.at[b], gsems[b]).wait()

    def store_out(t, b):
        s, b0 = tile_coords(t)
        pltpu.async_copy(outbuf.at[b], out_hbm.at[s, pl.ds(b0, C)], osems[b])

    def wait_store(t, b):
        s, b0 = tile_coords(t)
        pltpu.make_async_copy(outbuf.at[b], out_hbm.at[s, pl.ds(b0, C)],
                              osems[b]).wait()

    fetch_idx(0, 0)
    wait_idx(0, 0)
    gather(0, 0)
    fetch_idx(1, 1)
    wait_idx(1, 1)
    gather(1, 1)

    def pair(i, carry):
        for b in range(2):
            t = 2 * i + b
            s, b0 = tile_coords(t)

            wait_gather(t, b)

            @pl.when(t + 2 < PER_W)
            def _():
                fetch_idx(t + 2, b)

            @pl.when(t >= 2)
            def _():
                wait_store(t - 2, b)

            blk = s >> 7
            slocv = jnp.broadcast_to(s & 127, (NLANES,))
            p = [plsc.load_gather(posbuf.at[blk], [erow[j], slocv])
                 for j in range(NJ)]

            def addrows(k, c2):
                for u in range(ROW_UNROLL):
                    r = k * ROW_UNROLL + u
                    for j in range(NJ):
                        sl = pl.ds(j * NLANES, NLANES)
                        outbuf[b, r, sl] = rows[b, r, sl] + p[j]
                return c2

            lax.fori_loop(0, C // ROW_UNROLL, addrows, 0)

            store_out(t, b)

            @pl.when(t + 2 < PER_W)
            def _():
                wait_idx(t + 2, b)
                gather(t + 2, b)
        return carry

    lax.fori_loop(0, PER_W // 2, pair, 0)

    wait_store(PER_W - 2, 0)
    wait_store(PER_W - 1, 1)


def kernel(x, encoder, pos_emb):
    mesh = plsc.VectorSubcoreMesh(core_axis_name="c", subcore_axis_name="s")
    run = functools.partial(
        pl.kernel,
        mesh=mesh,
        out_type=jax.ShapeDtypeStruct((SEQ, BATCH, EMB), jnp.float32),
        scratch_types=[
            pltpu.VMEM((2, C), jnp.int32),
            pltpu.VMEM((2, C, 2 * EMB), jnp.float32),
            pltpu.VMEM((2, EMB, 128), jnp.float32),
            pltpu.VMEM((2, C, EMB), jnp.float32),
            pltpu.SemaphoreType.DMA,
            pltpu.SemaphoreType.DMA,
            pltpu.SemaphoreType.DMA,
            pltpu.SemaphoreType.DMA,
            pltpu.SemaphoreType.DMA,
            pltpu.SemaphoreType.DMA,
        ],
        compiler_params=pltpu.CompilerParams(
            use_tc_tiling_on_sc=True, needs_layout_passes=False),
    )(_emb_kernel)
    enc_padded = jnp.pad(encoder, ((0, 0), (0, 2 * EMB - EMB)))
    return run(x, enc_padded, pos_emb.T)
